# baseline (speedup 1.0000x reference)
import functools
import jax
import jax.numpy as jnp
from jax import lax
from jax.experimental import pallas as pl
from jax.experimental.pallas import tpu as pltpu
from jax.experimental.pallas import tpu_sc as plsc

NC, NS, LANES = 2, 16, 16
NW = NC * NS
NXB = 3


def kernel(x, table):
    B, L, D = x.shape
    rows_per_w = L // NW
    CH = 32
    n_ch = rows_per_w // CH
    nvr = D // LANES
    NU = n_ch * B

    mesh = plsc.VectorSubcoreMesh(core_axis_name="c", subcore_axis_name="s")

    @functools.partial(
        pl.kernel,
        out_type=jax.ShapeDtypeStruct((B * L, D), jnp.float32),
        mesh=mesh,
        scratch_types=[
            [pltpu.VMEM((CH, D), jnp.float32) for _ in range(2)],
            [pltpu.VMEM((CH, D), jnp.float32) for _ in range(NXB)],
            [pltpu.SemaphoreType.DMA for _ in range(2)],
            [pltpu.SemaphoreType.DMA for _ in range(NXB)],
            [pltpu.SemaphoreType.DMA for _ in range(NXB)],
        ],
    )
    def sc_add(x_hbm, t_hbm, o_hbm, tbufs, xbufs, tsems, xsems, osems):
        wid = lax.axis_index("s") * NC + lax.axis_index("c")
        base = wid * rows_per_w

        def t_load(c):
            return pltpu.async_copy(
                t_hbm.at[pl.ds(base + c * CH, CH), :], tbufs[c % 2], tsems[c % 2])

        def x_row(u):
            c, b = divmod(u, B)
            return b * L + base + c * CH

        def x_load(u):
            return pltpu.async_copy(
                x_hbm.at[pl.ds(x_row(u), CH), :], xbufs[u % NXB], xsems[u % NXB])

        def o_store(u):
            return pltpu.async_copy(
                xbufs[u % NXB], o_hbm.at[pl.ds(x_row(u), CH), :], osems[u % NXB])

        ost = {}
        waited = set()

        def wait_o(u):
            if u >= 0 and u not in waited:
                ost[u].wait()
                waited.add(u)

        tld = {0: t_load(0)}
        if n_ch > 1:
            tld[1] = t_load(1)
        xld = {u: x_load(u) for u in range(min(NXB - 1, NU))}
        for u in range(NU):
            c, b = divmod(u, B)
            xld[u].wait()
            if b == 0:
                tld[c].wait()
            xb = xbufs[u % NXB]
            tb = tbufs[c % 2]

            @plsc.parallel_loop(0, CH * nvr, unroll=16)
            def _(i):
                r = lax.bitwise_and(i, CH - 1)
                d0 = lax.shift_right_logical(i, CH.bit_length() - 1) * LANES
                sl = pl.ds(d0, LANES)
                tv = tb[r, sl]
                plsc.addupdate(xb.at[r, sl], tv)

            ost[u] = o_store(u)
            if b == B - 1 and c + 2 < n_ch:
                tld[c + 2] = t_load(c + 2)
            if u + NXB - 1 < NU:
                wait_o(u - 1)
                xld[u + NXB - 1] = x_load(u + NXB - 1)
        for u in range(max(0, NU - NXB), NU):
            wait_o(u)

    out = sc_add(x.reshape(B * L, D), table)
    return out.reshape(B, L, D)


# speedup vs baseline: 1.0811x; 1.0811x over previous
import functools
import jax
import jax.numpy as jnp
from jax import lax
from jax.experimental import pallas as pl
from jax.experimental.pallas import tpu as pltpu
from jax.experimental.pallas import tpu_sc as plsc

NC, NS, LANES = 2, 16, 16
NW = NC * NS
NXB = 6
LEAD = 4


def kernel(x, table):
    B, L, D = x.shape
    rows_per_w = L // NW
    CH = 16
    n_ch = rows_per_w // CH
    nvr = D // LANES
    NU = n_ch * B

    mesh = plsc.VectorSubcoreMesh(core_axis_name="c", subcore_axis_name="s")

    @functools.partial(
        pl.kernel,
        out_type=jax.ShapeDtypeStruct((B * L, D), jnp.float32),
        mesh=mesh,
        scratch_types=[
            [pltpu.VMEM((CH, D), jnp.float32) for _ in range(2)],
            [pltpu.VMEM((CH, D), jnp.float32) for _ in range(NXB)],
            [pltpu.SemaphoreType.DMA for _ in range(2)],
            [pltpu.SemaphoreType.DMA for _ in range(NXB)],
            [pltpu.SemaphoreType.DMA for _ in range(NXB)],
        ],
    )
    def sc_add(x_hbm, t_hbm, o_hbm, tbufs, xbufs, tsems, xsems, osems):
        wid = lax.axis_index("s") * NC + lax.axis_index("c")
        base = wid * rows_per_w

        def t_load(c):
            return pltpu.async_copy(
                t_hbm.at[pl.ds(base + c * CH, CH), :], tbufs[c % 2], tsems[c % 2])

        def x_row(u):
            c, b = divmod(u, B)
            return b * L + base + c * CH

        def x_load(u):
            return pltpu.async_copy(
                x_hbm.at[pl.ds(x_row(u), CH), :], xbufs[u % NXB], xsems[u % NXB])

        def o_store(u):
            return pltpu.async_copy(
                xbufs[u % NXB], o_hbm.at[pl.ds(x_row(u), CH), :], osems[u % NXB])

        ost = {}
        waited = set()

        def wait_o(u):
            if u >= 0 and u not in waited:
                ost[u].wait()
                waited.add(u)

        tld = {0: t_load(0)}
        if n_ch > 1:
            tld[1] = t_load(1)
        xld = {u: x_load(u) for u in range(min(LEAD, NU))}
        for u in range(NU):
            c, b = divmod(u, B)
            xld[u].wait()
            if b == 0:
                tld[c].wait()
            xb = xbufs[u % NXB]
            tb = tbufs[c % 2]

            @plsc.parallel_loop(0, CH * nvr, unroll=8)
            def _(i):
                r = lax.bitwise_and(i, CH - 1)
                d0 = lax.shift_right_logical(i, CH.bit_length() - 1) * LANES
                sl = pl.ds(d0, LANES)
                tv = tb[r, sl]
                plsc.addupdate(xb.at[r, sl], tv)

            ost[u] = o_store(u)
            if b == B - 1 and c + 2 < n_ch:
                tld[c + 2] = t_load(c + 2)
            if u + LEAD < NU:
                wait_o(u + LEAD - NXB)
                xld[u + LEAD] = x_load(u + LEAD)
        for u in range(max(0, NU - NXB), NU):
            wait_o(u)

    out = sc_add(x.reshape(B * L, D), table)
    return out.reshape(B, L, D)
